# Initial kernel scaffold; baseline (speedup 1.0000x reference)
import jax
import jax.numpy as jnp
from jax.experimental import pallas as pl


def kernel(pose, SH_, normal, params):
    raise NotImplementedError("write your pallas kernel here")


# speedup vs baseline: 10.4126x; 10.4126x over previous
import functools

import jax
import jax.numpy as jnp
from jax import lax
from jax.experimental import pallas as pl
from jax.experimental.pallas import tpu as pltpu
from jax.experimental.pallas import tpu_sc as plsc

_B, _N, _K1 = 8, 2048, 16


def _sc_gather(table, idx, chunk):
    V, D = table.shape
    Btot = idx.shape[0]
    info = plsc.get_sparse_core_info()
    NC, NS = info.num_cores, info.num_subcores
    NW = NC * NS
    b_per_w = Btot // NW
    assert Btot == b_per_w * NW and b_per_w % chunk == 0
    nchunks = b_per_w // chunk
    mesh = plsc.VectorSubcoreMesh(core_axis_name="c", subcore_axis_name="s")

    @functools.partial(
        pl.kernel,
        mesh=mesh,
        out_type=jax.ShapeDtypeStruct((Btot, D), jnp.float32),
        compiler_params=pltpu.CompilerParams(use_tc_tiling_on_sc=False),
        scratch_types=[
            pltpu.VMEM((chunk,), jnp.int32),
            pltpu.VMEM((chunk, D), jnp.float32),
            pltpu.SemaphoreType.DMA,
        ],
    )
    def k(table_hbm, idx_hbm, out_hbm, idx_v, rows_v, sem):
        wid = lax.axis_index("s") * NC + lax.axis_index("c")
        for c in range(nchunks):
            base = wid * b_per_w + c * chunk
            pltpu.sync_copy(idx_hbm.at[pl.ds(base, chunk)], idx_v)
            pltpu.async_copy(table_hbm.at[idx_v], rows_v, sem).wait()
            pltpu.sync_copy(rows_v, out_hbm.at[pl.ds(base, chunk)])

    return k(table, idx)


def _acos(x):
    ax = jnp.abs(x)
    p = -0.0012624911
    for c in (0.0066700901, -0.0170881256, 0.0308918810, -0.0501743046,
              0.0889789874, -0.2145988016, 1.5707963050):
        p = p * ax + c
    r = jnp.sqrt(jnp.maximum(1.0 - ax, 0.0)) * p
    return jnp.where(x >= 0.0, r, jnp.float32(3.14159265358979) - r)


def _angle(a, b):
    na = jnp.sqrt(jnp.sum(a * a, -1))
    nb = jnp.sqrt(jnp.sum(b * b, -1))
    denom = na * nb
    cos = jnp.sum(a * b, -1) / jnp.maximum(denom, 1e-8)
    ang = _acos(jnp.clip(cos, -1.0 + 1e-6, 1.0 - 1e-6))
    return jnp.where(denom > 1e-8, ang, 0.0)


def _sqdist_block(q, pt):
    acc = None
    for c in range(3):
        diff = q[:, c:c + 1] - pt[c:c + 1, :]
        sq = diff * diff
        acc = sq if acc is None else acc + sq
    return acc


def _extract_knn(d2, nk):
    Q, Ncand = d2.shape
    iota = lax.broadcasted_iota(jnp.int32, (Q, Ncand), 1)
    cols = []
    for _ in range(nk):
        m = jnp.min(d2, axis=1, keepdims=True)
        amin = jnp.min(jnp.where(d2 == m, iota, Ncand), axis=1, keepdims=True)
        cols.append(amin)
        d2 = jnp.where(iota == amin, jnp.inf, d2)
    return cols


def _mlp_block(x, wbs):
    for w, b in wbs:
        x = jnp.maximum(jnp.dot(x, w, preferred_element_type=jnp.float32)
                        + b, 0.0)
    return x


def _full(shape):
    return pl.BlockSpec(shape, lambda *a: (0,) * len(shape))


def _knn(q_pts, pose_t, nk, qblk):
    Bb, Nq, _ = q_pts.shape
    Ncand = pose_t.shape[2]

    def body(q_ref, pt_ref, o_ref):
        b = pl.program_id(0)
        d2 = _sqdist_block(q_ref[0], pt_ref[0])
        cols = _extract_knn(d2, nk)
        cols = cols + [cols[0]] * (_K1 - nk)
        o_ref[0] = jnp.concatenate(cols, axis=1) + b * Ncand

    return pl.pallas_call(
        body,
        grid=(Bb, Nq // qblk),
        in_specs=[
            pl.BlockSpec((1, qblk, 3), lambda b, q: (b, q, 0)),
            pl.BlockSpec((1, 3, Ncand), lambda b, q: (b, 0, 0)),
        ],
        out_specs=pl.BlockSpec((1, qblk, _K1), lambda b, q: (b, q, 0)),
        out_shape=jax.ShapeDtypeStruct((Bb, Nq, _K1), jnp.int32),
    )(q_pts, pose_t)


def _spf_rri(g, pose, normal, sh, rri, qblk):
    (w0, b0), (w1, b1) = rri

    def body(g_ref, p_ref, n_ref, pfull_ref, sh_ref, w0_ref, b0_ref,
             w1_ref, b1_ref, o_ref):
        gb = g_ref[0]
        p = p_ref[0]
        nc = n_ref[0]
        nbr_p = gb[:, :, 0:3]
        nbr_n = gb[:, :, 3:6]
        d = nbr_p - p[:, None, :]
        dist = jnp.sqrt(jnp.sum(d * d, -1))
        n_cb = jnp.broadcast_to(nc[:, None, :], d.shape)
        a1 = _angle(n_cb, d)
        a2 = _angle(nbr_n, d)
        a3 = _angle(n_cb, nbr_n)
        kio = lax.broadcasted_iota(jnp.int32, dist.shape, 1)
        msk = (kio >= 1).astype(jnp.float32)
        spf = jnp.stack(
            [jnp.sum(a1 * msk, 1), jnp.sum(a2 * msk, 1),
             jnp.sum(a3 * msk, 1), jnp.sum(dist * msk, 1)], axis=-1
        ) * (1.0 / 15.0)
        pf = pfull_ref[0]
        mean = jnp.sum(pf, axis=0, keepdims=True) * (1.0 / pf.shape[0])
        cd = p - mean
        pose_dis = jnp.sqrt(jnp.sum(cd * cd, -1, keepdims=True))
        x = jnp.concatenate([spf, sh_ref[0], pose_dis], axis=-1)
        o_ref[0] = _mlp_block(x, [(w0_ref[...], b0_ref[...]),
                                  (w1_ref[...], b1_ref[...])])

    return pl.pallas_call(
        body,
        grid=(_B, _N // qblk),
        in_specs=[
            pl.BlockSpec((1, qblk, _K1, 16), lambda b, q: (b, q, 0, 0)),
            pl.BlockSpec((1, qblk, 3), lambda b, q: (b, q, 0)),
            pl.BlockSpec((1, qblk, 3), lambda b, q: (b, q, 0)),
            pl.BlockSpec((1, _N, 3), lambda b, q: (b, 0, 0)),
            pl.BlockSpec((1, qblk, 10), lambda b, q: (b, q, 0)),
            _full(w0.shape), _full((1, 64)), _full(w1.shape), _full((1, 64)),
        ],
        out_specs=pl.BlockSpec((1, qblk, 64), lambda b, q: (b, q, 0)),
        out_shape=jax.ShapeDtypeStruct((_B, _N, 64), jnp.float32),
    )(g, pose, normal, pose, sh, w0, b0.reshape(1, -1), w1, b1.reshape(1, -1))


def _sa(g, q_pts, layers, cin, qblk):
    Bb, Nq = g.shape[0], g.shape[1]
    dpad = g.shape[3]
    wbs = [(w, b.reshape(1, -1)) for (w, b) in layers]
    cout = wbs[-1][0].shape[1]

    def body(g_ref, q_ref, *refs):
        o_ref = refs[-1]
        wb_refs = [(refs[2 * i][...], refs[2 * i + 1][...])
                   for i in range(len(wbs))]
        gb = g_ref[0]
        q = q_ref[0]
        rel = gb[:, :, 0:3] - q[:, None, :]
        x = jnp.concatenate([rel, gb[:, :, 3:3 + cin]], axis=-1)
        x = x.reshape(qblk * _K1, 3 + cin)
        h = _mlp_block(x, wb_refs)
        h = h.reshape(qblk, _K1, cout)
        o_ref[0] = jnp.max(h, axis=1)

    flat = []
    for w, b in wbs:
        flat += [w, b]
    return pl.pallas_call(
        body,
        grid=(Bb, Nq // qblk),
        in_specs=[
            pl.BlockSpec((1, qblk, _K1, dpad), lambda b, q: (b, q, 0, 0)),
            pl.BlockSpec((1, qblk, 3), lambda b, q: (b, q, 0)),
        ] + [_full(a.shape) for a in flat],
        out_specs=pl.BlockSpec((1, qblk, cout), lambda b, q: (b, q, 0)),
        out_shape=jax.ShapeDtypeStruct((Bb, Nq, cout), jnp.float32),
    )(g, q_pts, *flat)


def _tail(feat2, sh2, pose2, params):
    grri = [(w, b.reshape(1, -1)) for (w, b) in params["grri"]]
    gsa = [(w, b.reshape(1, -1)) for (w, b) in params["gsa"]]
    w1, b1 = params["lin1"]
    g1, be1 = params["bn1"]
    w2, b2 = params["lin2"]
    g2, be2 = params["bn2"]
    w3, b3 = params["lin3"]
    head = [w1, b1.reshape(1, -1), g1.reshape(1, -1), be1.reshape(1, -1),
            w2, b2.reshape(1, -1), g2.reshape(1, -1), be2.reshape(1, -1),
            w3, b3.reshape(1, -1)]
    ng, nsa = len(grri), len(gsa)
    flat = []
    for w, b in grri + gsa:
        flat += [w, b]
    flat += head
    npts = feat2.shape[1]

    def body(f_ref, sh_ref, p_ref, *refs):
        o_ref = refs[-1]
        vals = [r[...] for r in refs[:-1]]
        wb = [(vals[2 * i], vals[2 * i + 1]) for i in range(ng + nsa)]
        (hw1, hb1, hg1, hbe1, hw2, hb2, hg2, hbe2, hw3, hb3) = \
            vals[2 * (ng + nsa):]
        p = p_ref[0]
        mean = jnp.sum(p, axis=0, keepdims=True) * (1.0 / npts)
        cd = p - mean
        pose_dis = jnp.sqrt(jnp.sum(cd * cd, -1, keepdims=True))
        x0 = jnp.concatenate(
            [jnp.zeros((npts, 4), jnp.float32), sh_ref[0], pose_dis], -1)
        gfe = _mlp_block(x0, wb[:ng])
        h = jnp.concatenate([f_ref[0], gfe], axis=-1)
        h = _mlp_block(h, wb[ng:])
        x = jnp.max(h, axis=0, keepdims=True)
        x = jnp.maximum(
            (jnp.dot(x, hw1, preferred_element_type=jnp.float32) + hb1)
            * hg1 + hbe1, 0.0)
        x = jnp.maximum(
            (jnp.dot(x, hw2, preferred_element_type=jnp.float32) + hb2)
            * hg2 + hbe2, 0.0)
        x = jnp.dot(x, hw3, preferred_element_type=jnp.float32) + hb3
        m = jnp.max(x, axis=-1, keepdims=True)
        e = jnp.exp(x - m)
        o_ref[0] = x - m - jnp.log(jnp.sum(e, axis=-1, keepdims=True))

    out = pl.pallas_call(
        body,
        grid=(_B,),
        in_specs=[
            pl.BlockSpec((1, npts, 512), lambda b: (b, 0, 0)),
            pl.BlockSpec((1, npts, 10), lambda b: (b, 0, 0)),
            pl.BlockSpec((1, npts, 3), lambda b: (b, 0, 0)),
        ] + [_full(a.shape) for a in flat],
        out_specs=pl.BlockSpec((1, 1, 40), lambda b: (b, 0, 0)),
        out_shape=jax.ShapeDtypeStruct((_B, 1, 40), jnp.float32),
    )(feat2, sh2, pose2, *flat)
    return out.reshape(_B, 40)


def _pad_table(*parts):
    x = jnp.concatenate(parts, axis=-1)
    c = x.shape[-1]
    cpad = -(-c // 16) * 16
    if cpad != c:
        x = jnp.concatenate(
            [x, jnp.zeros(x.shape[:-1] + (cpad - c,), jnp.float32)], -1)
    return x.reshape(-1, cpad)


def kernel(pose, SH_, normal, params):
    pose_t = jnp.transpose(pose, (0, 2, 1))

    idx0 = _knn(pose, pose_t, 16, 256)
    tbl0 = _pad_table(pose, normal)
    g0 = _sc_gather(tbl0, idx0.reshape(-1), 2048)
    g0 = g0.reshape(_B, _N, _K1, 16)
    feature0 = _spf_rri(g0, pose, normal, SH_, params["rri"], 256)

    pose_s1 = pose[:, ::2]
    idx1 = _knn(pose_s1, pose_t, 15, 256)
    tbl1 = _pad_table(pose, feature0)
    g1 = _sc_gather(tbl1, idx1.reshape(-1), 1024)
    g1 = g1.reshape(_B, 1024, _K1, 80)
    feat1 = _sa(g1, pose_s1, params["sa1"], 64, 256)

    pose1_t = jnp.transpose(pose_s1, (0, 2, 1))
    pose_s2 = pose[:, ::8]
    idx2 = _knn(pose_s2, pose1_t, 15, 256)
    tbl2 = _pad_table(pose_s1, feat1)
    g2 = _sc_gather(tbl2, idx2.reshape(-1), 256)
    g2 = g2.reshape(_B, 256, _K1, 272)
    feat2 = _sa(g2, pose_s2, params["sa2"], 256, 256)

    sh2 = SH_[:, ::8]
    return _tail(feat2, sh2, pose_s2, params)


# speedup vs baseline: 18.8201x; 1.8074x over previous
import functools

import jax
import jax.numpy as jnp
from jax import lax
from jax.experimental import pallas as pl
from jax.experimental.pallas import tpu as pltpu
from jax.experimental.pallas import tpu_sc as plsc

_B, _N, _K1 = 8, 2048, 16


def _sc_gather(table, idx, chunk):
    V, D = table.shape
    Btot = idx.shape[0]
    info = plsc.get_sparse_core_info()
    NC, NS = info.num_cores, info.num_subcores
    NW = NC * NS
    b_per_w = Btot // NW
    assert Btot == b_per_w * NW and b_per_w % chunk == 0
    nchunks = b_per_w // chunk
    mesh = plsc.VectorSubcoreMesh(core_axis_name="c", subcore_axis_name="s")

    @functools.partial(
        pl.kernel,
        mesh=mesh,
        out_type=jax.ShapeDtypeStruct((Btot, D), jnp.float32),
        compiler_params=pltpu.CompilerParams(use_tc_tiling_on_sc=False),
        scratch_types=[
            pltpu.VMEM((chunk,), jnp.int32),
            pltpu.VMEM((chunk, D), jnp.float32),
            pltpu.SemaphoreType.DMA,
        ],
    )
    def k(table_hbm, idx_hbm, out_hbm, idx_v, rows_v, sem):
        wid = lax.axis_index("s") * NC + lax.axis_index("c")
        for c in range(nchunks):
            base = wid * b_per_w + c * chunk
            pltpu.sync_copy(idx_hbm.at[pl.ds(base, chunk)], idx_v)
            pltpu.async_copy(table_hbm.at[idx_v], rows_v, sem).wait()
            pltpu.sync_copy(rows_v, out_hbm.at[pl.ds(base, chunk)])

    return k(table, idx)


def _acos(x):
    ax = jnp.abs(x)
    p = -0.0012624911
    for c in (0.0066700901, -0.0170881256, 0.0308918810, -0.0501743046,
              0.0889789874, -0.2145988016, 1.5707963050):
        p = p * ax + c
    r = jnp.sqrt(jnp.maximum(1.0 - ax, 0.0)) * p
    return jnp.where(x >= 0.0, r, jnp.float32(3.14159265358979) - r)


def _angle(a, b):
    na = jnp.sqrt(jnp.sum(a * a, -1))
    nb = jnp.sqrt(jnp.sum(b * b, -1))
    denom = na * nb
    cos = jnp.sum(a * b, -1) / jnp.maximum(denom, 1e-8)
    ang = _acos(jnp.clip(cos, -1.0 + 1e-6, 1.0 - 1e-6))
    return jnp.where(denom > 1e-8, ang, 0.0)


def _sqdist_block(q, pt):
    acc = None
    for c in range(3):
        diff = q[:, c:c + 1] - pt[c:c + 1, :]
        sq = diff * diff
        acc = sq if acc is None else acc + sq
    return acc


def _extract_knn(d2, nk):
    Q, Ncand = d2.shape
    iota = lax.broadcasted_iota(jnp.int32, (Q, Ncand), 1)
    packed = (lax.bitcast_convert_type(d2, jnp.int32)
              & jnp.int32(-2048)) | iota
    cols = []
    for _ in range(nk):
        m = jnp.min(packed, axis=1, keepdims=True)
        cols.append(m & 2047)
        packed = jnp.where(packed == m, jnp.int32(0x7FFFFFFF), packed)
    return cols


def _mlp_block(x, wbs):
    for w, b in wbs:
        x = jnp.maximum(jnp.dot(x, w, preferred_element_type=jnp.float32)
                        + b, 0.0)
    return x


def _full(shape):
    return pl.BlockSpec(shape, lambda *a: (0,) * len(shape))


def _knn(q_pts, pose_t, nk, qblk):
    Bb, Nq, _ = q_pts.shape
    Ncand = pose_t.shape[2]

    def body(q_ref, pt_ref, o_ref):
        b = pl.program_id(0)
        d2 = _sqdist_block(q_ref[0], pt_ref[0])
        cols = _extract_knn(d2, nk)
        cols = cols + [cols[0]] * (_K1 - nk)
        o_ref[0] = jnp.concatenate(cols, axis=1) + b * Ncand

    return pl.pallas_call(
        body,
        grid=(Bb, Nq // qblk),
        in_specs=[
            pl.BlockSpec((1, qblk, 3), lambda b, q: (b, q, 0)),
            pl.BlockSpec((1, 3, Ncand), lambda b, q: (b, 0, 0)),
        ],
        out_specs=pl.BlockSpec((1, qblk, _K1), lambda b, q: (b, q, 0)),
        out_shape=jax.ShapeDtypeStruct((Bb, Nq, _K1), jnp.int32),
    )(q_pts, pose_t)


def _ang_dense(dot, na, nb):
    denom = na * nb
    cos = dot / jnp.maximum(denom, 1e-8)
    ang = _acos(jnp.clip(cos, -1.0 + 1e-6, 1.0 - 1e-6))
    return jnp.where(denom > 1e-8, ang, 0.0)


def _spf_rri(a6, q6, pose, sh, rri):
    (w0, b0), (w1, b1) = rri

    def body(a_ref, qq_ref, p_ref, sh_ref, w0_ref, b0_ref,
             w1_ref, b1_ref, o_ref):
        A = a_ref[0]
        Q = qq_ref[0]
        d0 = A[0] - Q[0]
        d1 = A[1] - Q[1]
        d2_ = A[2] - Q[2]
        dist = jnp.sqrt(d0 * d0 + d1 * d1 + d2_ * d2_)
        nd = Q[3] * d0 + Q[4] * d1 + Q[5] * d2_
        nnd = A[3] * d0 + A[4] * d1 + A[5] * d2_
        nnn = Q[3] * A[3] + Q[4] * A[4] + Q[5] * A[5]
        na = jnp.sqrt(Q[3] * Q[3] + Q[4] * Q[4] + Q[5] * Q[5])
        nb = jnp.sqrt(A[3] * A[3] + A[4] * A[4] + A[5] * A[5])
        a1 = _ang_dense(nd, na, dist)
        a2 = _ang_dense(nnd, nb, dist)
        a3 = _ang_dense(nnn, na, nb)
        lio = lax.broadcasted_iota(jnp.int32, dist.shape, 1)
        msk = ((lio & 15) >= 1).astype(jnp.float32)
        feats = []
        for f in (a1, a2, a3, dist):
            s = jnp.sum((f * msk).reshape(256, 8, 16), axis=2) * (1.0 / 15.0)
            feats.append(s.reshape(_N, 1))
        p = p_ref[0]
        mean = jnp.sum(p, axis=0, keepdims=True) * (1.0 / _N)
        cd = p - mean
        pose_dis = jnp.sqrt(jnp.sum(cd * cd, -1, keepdims=True))
        x = jnp.concatenate(feats + [sh_ref[0], pose_dis], axis=-1)
        o_ref[0] = _mlp_block(x, [(w0_ref[...], b0_ref[...]),
                                  (w1_ref[...], b1_ref[...])])

    return pl.pallas_call(
        body,
        grid=(_B,),
        in_specs=[
            pl.BlockSpec((1, 6, 256, 128), lambda b: (b, 0, 0, 0)),
            pl.BlockSpec((1, 6, 256, 128), lambda b: (b, 0, 0, 0)),
            pl.BlockSpec((1, _N, 3), lambda b: (b, 0, 0)),
            pl.BlockSpec((1, _N, 10), lambda b: (b, 0, 0)),
            _full(w0.shape), _full((1, 64)), _full(w1.shape), _full((1, 64)),
        ],
        out_specs=pl.BlockSpec((1, _N, 64), lambda b: (b, 0, 0)),
        out_shape=jax.ShapeDtypeStruct((_B, _N, 64), jnp.float32),
    )(a6, q6, pose, sh, w0, b0.reshape(1, -1), w1, b1.reshape(1, -1))


def _sa(g, q_pts, layers, cin, qblk):
    Bb, Nq = g.shape[0], g.shape[1]
    dpad = g.shape[3]
    wbs = [(w, b.reshape(1, -1)) for (w, b) in layers]
    cout = wbs[-1][0].shape[1]

    def body(g_ref, q_ref, *refs):
        o_ref = refs[-1]
        wb_refs = [(refs[2 * i][...], refs[2 * i + 1][...])
                   for i in range(len(wbs))]
        gb = g_ref[0]
        q = q_ref[0]
        rel = gb[:, :, 0:3] - q[:, None, :]
        x = jnp.concatenate([rel, gb[:, :, 3:3 + cin]], axis=-1)
        x = x.reshape(qblk * _K1, 3 + cin)
        h = _mlp_block(x, wb_refs)
        h = h.reshape(qblk, _K1, cout)
        o_ref[0] = jnp.max(h, axis=1)

    flat = []
    for w, b in wbs:
        flat += [w, b]
    return pl.pallas_call(
        body,
        grid=(Bb, Nq // qblk),
        in_specs=[
            pl.BlockSpec((1, qblk, _K1, dpad), lambda b, q: (b, q, 0, 0)),
            pl.BlockSpec((1, qblk, 3), lambda b, q: (b, q, 0)),
        ] + [_full(a.shape) for a in flat],
        out_specs=pl.BlockSpec((1, qblk, cout), lambda b, q: (b, q, 0)),
        out_shape=jax.ShapeDtypeStruct((Bb, Nq, cout), jnp.float32),
    )(g, q_pts, *flat)


def _tail(feat2, sh2, pose2, params):
    grri = [(w, b.reshape(1, -1)) for (w, b) in params["grri"]]
    gsa = [(w, b.reshape(1, -1)) for (w, b) in params["gsa"]]
    w1, b1 = params["lin1"]
    g1, be1 = params["bn1"]
    w2, b2 = params["lin2"]
    g2, be2 = params["bn2"]
    w3, b3 = params["lin3"]
    head = [w1, b1.reshape(1, -1), g1.reshape(1, -1), be1.reshape(1, -1),
            w2, b2.reshape(1, -1), g2.reshape(1, -1), be2.reshape(1, -1),
            w3, b3.reshape(1, -1)]
    ng, nsa = len(grri), len(gsa)
    flat = []
    for w, b in grri + gsa:
        flat += [w, b]
    flat += head
    npts = feat2.shape[1]

    def body(f_ref, sh_ref, p_ref, *refs):
        o_ref = refs[-1]
        vals = [r[...] for r in refs[:-1]]
        wb = [(vals[2 * i], vals[2 * i + 1]) for i in range(ng + nsa)]
        (hw1, hb1, hg1, hbe1, hw2, hb2, hg2, hbe2, hw3, hb3) = \
            vals[2 * (ng + nsa):]
        p = p_ref[0]
        mean = jnp.sum(p, axis=0, keepdims=True) * (1.0 / npts)
        cd = p - mean
        pose_dis = jnp.sqrt(jnp.sum(cd * cd, -1, keepdims=True))
        x0 = jnp.concatenate(
            [jnp.zeros((npts, 4), jnp.float32), sh_ref[0], pose_dis], -1)
        gfe = _mlp_block(x0, wb[:ng])
        h = jnp.concatenate([f_ref[0], gfe], axis=-1)
        h = _mlp_block(h, wb[ng:])
        x = jnp.max(h, axis=0, keepdims=True)
        x = jnp.maximum(
            (jnp.dot(x, hw1, preferred_element_type=jnp.float32) + hb1)
            * hg1 + hbe1, 0.0)
        x = jnp.maximum(
            (jnp.dot(x, hw2, preferred_element_type=jnp.float32) + hb2)
            * hg2 + hbe2, 0.0)
        x = jnp.dot(x, hw3, preferred_element_type=jnp.float32) + hb3
        m = jnp.max(x, axis=-1, keepdims=True)
        e = jnp.exp(x - m)
        o_ref[0] = x - m - jnp.log(jnp.sum(e, axis=-1, keepdims=True))

    out = pl.pallas_call(
        body,
        grid=(_B,),
        in_specs=[
            pl.BlockSpec((1, npts, 512), lambda b: (b, 0, 0)),
            pl.BlockSpec((1, npts, 10), lambda b: (b, 0, 0)),
            pl.BlockSpec((1, npts, 3), lambda b: (b, 0, 0)),
        ] + [_full(a.shape) for a in flat],
        out_specs=pl.BlockSpec((1, 1, 40), lambda b: (b, 0, 0)),
        out_shape=jax.ShapeDtypeStruct((_B, 1, 40), jnp.float32),
    )(feat2, sh2, pose2, *flat)
    return out.reshape(_B, 40)


def _pad_table(*parts):
    x = jnp.concatenate(parts, axis=-1)
    c = x.shape[-1]
    cpad = -(-c // 16) * 16
    if cpad != c:
        x = jnp.concatenate(
            [x, jnp.zeros(x.shape[:-1] + (cpad - c,), jnp.float32)], -1)
    return x.reshape(-1, cpad)


def kernel(pose, SH_, normal, params):
    pose_t = jnp.transpose(pose, (0, 2, 1))

    idx0 = _knn(pose, pose_t, 16, 256)
    tbl0 = _pad_table(pose, normal)
    g0 = _sc_gather(tbl0, idx0.reshape(-1), 2048)
    g0 = g0.reshape(_B, _N, _K1, 16)
    a6 = jnp.transpose(g0[..., :6], (0, 3, 1, 2)).reshape(_B, 6, 256, 128)
    qcat = jnp.concatenate([pose, normal], -1)
    q6 = jnp.repeat(
        jnp.transpose(qcat, (0, 2, 1)).reshape(_B, 6, 256, 8), 16, axis=3)
    feature0 = _spf_rri(a6, q6, pose, SH_, params["rri"])

    pose_s1 = pose[:, ::2]
    idx1 = _knn(pose_s1, pose_t, 15, 256)
    tbl1 = _pad_table(pose, feature0)
    g1 = _sc_gather(tbl1, idx1.reshape(-1), 1024)
    g1 = g1.reshape(_B, 1024, _K1, 80)
    feat1 = _sa(g1, pose_s1, params["sa1"], 64, 256)

    pose1_t = jnp.transpose(pose_s1, (0, 2, 1))
    pose_s2 = pose[:, ::8]
    idx2 = _knn(pose_s2, pose1_t, 15, 256)
    tbl2 = _pad_table(pose_s1, feat1)
    g2 = _sc_gather(tbl2, idx2.reshape(-1), 256)
    g2 = g2.reshape(_B, 256, _K1, 272)
    feat2 = _sa(g2, pose_s2, params["sa2"], 256, 256)

    sh2 = SH_[:, ::8]
    return _tail(feat2, sh2, pose_s2, params)
